# baseline (speedup 1.0000x reference)
import jax
import jax.numpy as jnp
from jax.experimental import pallas as pl
from jax.experimental.pallas import tpu as pltpu

H, W, C = 512, 512, 3
S = 8
QD, KD = 32, 32
TOPK = 64
NP = (H // S) * (W // S)
PDIM = S * S * C
RB = 1024
NBLK = NP // RB
LOG2E = 1.4426950408889634


def _fused_kernel(patches_ref, wq_ref, bq_ref, wk_ref, bk_ref,
                  colsum_ref, bests_ref, idx_ref, k_scratch):
    i = pl.program_id(0)

    @pl.when(i == 0)
    def _init():
        k_scratch[...] = (
            jnp.dot(patches_ref[...], wk_ref[...],
                    preferred_element_type=jnp.float32) + bk_ref[...])
        colsum_ref[...] = jnp.zeros_like(colsum_ref)

    p_blk = patches_ref[pl.ds(i * RB, RB), :]
    q = (jnp.dot(p_blk, wq_ref[...], preferred_element_type=jnp.float32)
         + bq_ref[...])
    s = jax.lax.dot_general(
        q, k_scratch[...], (((1,), (1,)), ((), ())),
        preferred_element_type=jnp.float32)
    m = jnp.max(s, axis=1, keepdims=True)
    p = jnp.exp(s - m)
    z = jnp.sum(p, axis=1, keepdims=True)
    colsum_ref[...] += jnp.sum(p / z, axis=0, keepdims=True)

    @pl.when(i == NBLK - 1)
    def _topk():
        FR = 32
        FC = NP // FR
        cur = colsum_ref[...].reshape(FR, FC)
        gidx = (jax.lax.broadcasted_iota(jnp.int32, (FR, FC), 0) * FC
                + jax.lax.broadcasted_iota(jnp.int32, (FR, FC), 1))
        tlanes = jax.lax.broadcasted_iota(jnp.int32, (1, TOPK), 1)
        bvals = jnp.zeros((1, TOPK), jnp.float32)
        bidx = jnp.zeros((1, TOPK), jnp.int32)
        for t in range(TOPK):
            m1 = jnp.max(cur, axis=1, keepdims=True)
            am1 = jnp.min(jnp.where(cur == m1, gidx, NP),
                          axis=1, keepdims=True)
            mval = jnp.max(m1, axis=0, keepdims=True)
            midx = jnp.min(jnp.where(m1 == mval, am1, NP),
                           axis=0, keepdims=True)
            bvals = jnp.where(tlanes == t, mval, bvals)
            bidx = jnp.where(tlanes == t, midx, bidx)
            cur = jnp.where(gidx == midx, -jnp.inf, cur)
        bests_ref[...] = bvals
        idx_ref[...] = bidx


def kernel(input, Wq, bq, Wk, bk):
    patches = input.reshape(H // S, S, W // S, S * C)
    patches = patches.transpose(0, 2, 1, 3).reshape(NP, PDIM)
    colsum, bests, idx = pl.pallas_call(
        _fused_kernel,
        grid=(NBLK,),
        in_specs=[
            pl.BlockSpec((NP, PDIM), lambda i: (0, 0)),
            pl.BlockSpec((PDIM, QD), lambda i: (0, 0)),
            pl.BlockSpec((1, QD), lambda i: (0, 0)),
            pl.BlockSpec((PDIM, KD), lambda i: (0, 0)),
            pl.BlockSpec((1, KD), lambda i: (0, 0)),
        ],
        out_specs=[
            pl.BlockSpec((1, NP), lambda i: (0, 0)),
            pl.BlockSpec((1, TOPK), lambda i: (0, 0)),
            pl.BlockSpec((1, TOPK), lambda i: (0, 0)),
        ],
        out_shape=[
            jax.ShapeDtypeStruct((1, NP), jnp.float32),
            jax.ShapeDtypeStruct((1, TOPK), jnp.float32),
            jax.ShapeDtypeStruct((1, TOPK), jnp.int32),
        ],
        scratch_shapes=[pltpu.VMEM((NP, KD), jnp.float32)],
    )(patches, Wq, bq.reshape(1, QD), Wk, bk.reshape(1, KD))
    return bests[0], idx[0], colsum[0]


# speedup vs baseline: 1.0001x; 1.0001x over previous
import jax
import jax.numpy as jnp
from jax.experimental import pallas as pl
from jax.experimental.pallas import tpu as pltpu

H, W, C = 512, 512, 3
S = 8
QD, KD = 32, 32
TOPK = 64
NP = (H // S) * (W // S)
PDIM = S * S * C
RB = 1024
NBLK = NP // RB
LOG2E = 1.4426950408889634


def _fused_kernel(patches_ref, wq_ref, bq_ref, wk_ref, bk_ref,
                  colsum_ref, bests_ref, idx_ref, k_scratch):
    i = pl.program_id(0)

    @pl.when(i == 0)
    def _init():
        k_scratch[...] = (
            jnp.dot(patches_ref[...], wk_ref[...],
                    preferred_element_type=jnp.float32) + bk_ref[...])
        colsum_ref[...] = jnp.zeros_like(colsum_ref)

    p_blk = patches_ref[pl.ds(i * RB, RB), :]
    q = (jnp.dot(p_blk, wq_ref[...], preferred_element_type=jnp.float32)
         + bq_ref[...])
    s = jax.lax.dot_general(
        q, k_scratch[...], (((1,), (1,)), ((), ())),
        preferred_element_type=jnp.float32)
    m = jnp.max(s, axis=1, keepdims=True)
    p = jnp.exp(s - m)
    z = jnp.sum(p, axis=1, keepdims=True)
    colsum_ref[...] += jnp.sum(p / z, axis=0, keepdims=True)

    @pl.when(i == NBLK - 1)
    def _topk():
        FR = 8
        FC = NP // FR
        cur = colsum_ref[...].reshape(FR, FC)
        gidx = (jax.lax.broadcasted_iota(jnp.int32, (FR, FC), 0) * FC
                + jax.lax.broadcasted_iota(jnp.int32, (FR, FC), 1))
        tlanes = jax.lax.broadcasted_iota(jnp.int32, (1, TOPK), 1)
        bvals = jnp.zeros((1, TOPK), jnp.float32)
        bidx = jnp.zeros((1, TOPK), jnp.int32)
        for t in range(TOPK):
            m1 = jnp.max(cur, axis=1, keepdims=True)
            am1 = jnp.min(jnp.where(cur == m1, gidx, NP),
                          axis=1, keepdims=True)
            mval = jnp.max(m1, axis=0, keepdims=True)
            midx = jnp.min(jnp.where(m1 == mval, am1, NP),
                           axis=0, keepdims=True)
            bvals = jnp.where(tlanes == t, mval, bvals)
            bidx = jnp.where(tlanes == t, midx, bidx)
            cur = jnp.where(gidx == midx, -jnp.inf, cur)
        bests_ref[...] = bvals
        idx_ref[...] = bidx


def kernel(input, Wq, bq, Wk, bk):
    patches = input.reshape(H // S, S, W // S, S * C)
    patches = patches.transpose(0, 2, 1, 3).reshape(NP, PDIM)
    colsum, bests, idx = pl.pallas_call(
        _fused_kernel,
        grid=(NBLK,),
        in_specs=[
            pl.BlockSpec((NP, PDIM), lambda i: (0, 0)),
            pl.BlockSpec((PDIM, QD), lambda i: (0, 0)),
            pl.BlockSpec((1, QD), lambda i: (0, 0)),
            pl.BlockSpec((PDIM, KD), lambda i: (0, 0)),
            pl.BlockSpec((1, KD), lambda i: (0, 0)),
        ],
        out_specs=[
            pl.BlockSpec((1, NP), lambda i: (0, 0)),
            pl.BlockSpec((1, TOPK), lambda i: (0, 0)),
            pl.BlockSpec((1, TOPK), lambda i: (0, 0)),
        ],
        out_shape=[
            jax.ShapeDtypeStruct((1, NP), jnp.float32),
            jax.ShapeDtypeStruct((1, TOPK), jnp.float32),
            jax.ShapeDtypeStruct((1, TOPK), jnp.int32),
        ],
        scratch_shapes=[pltpu.VMEM((NP, KD), jnp.float32)],
    )(patches, Wq, bq.reshape(1, QD), Wk, bk.reshape(1, KD))
    return bests[0], idx[0], colsum[0]


# speedup vs baseline: 1.0280x; 1.0279x over previous
import jax
import jax.numpy as jnp
from jax.experimental import pallas as pl
from jax.experimental.pallas import tpu as pltpu

H, W, C = 512, 512, 3
S = 8
QD, KD = 32, 32
TOPK = 64
NP = (H // S) * (W // S)
PDIM = S * S * C
RB = 1024
NBLK = NP // RB
LOG2E = 1.4426950408889634


def _fused_kernel(patches_ref, wq_ref, bq_ref, wk_ref, bk_ref,
                  colsum_ref, bests_ref, idx_ref, k_scratch):
    i = pl.program_id(0)

    @pl.when(i == 0)
    def _init():
        k_scratch[...] = (
            jnp.dot(patches_ref[...], wk_ref[...],
                    preferred_element_type=jnp.float32) + bk_ref[...])
        colsum_ref[...] = jnp.zeros_like(colsum_ref)

    p_blk = patches_ref[pl.ds(i * RB, RB), :]
    q = (jnp.dot(p_blk, wq_ref[...], preferred_element_type=jnp.float32)
         + bq_ref[...])
    s = jax.lax.dot_general(
        q, k_scratch[...], (((1,), (1,)), ((), ())),
        preferred_element_type=jnp.float32)
    kk = k_scratch[...]
    mk2 = jnp.max(jnp.sum(kk * kk, axis=1, keepdims=True))
    b = jnp.sqrt(jnp.sum(q * q, axis=1, keepdims=True) * mk2)
    p = jnp.exp(s - b)
    z = jnp.sum(p, axis=1, keepdims=True)
    colsum_ref[...] += jnp.sum(p / z, axis=0, keepdims=True)

    @pl.when(i == NBLK - 1)
    def _topk():
        FR = 8
        FC = NP // FR
        cur = colsum_ref[...].reshape(FR, FC)
        gidx = (jax.lax.broadcasted_iota(jnp.int32, (FR, FC), 0) * FC
                + jax.lax.broadcasted_iota(jnp.int32, (FR, FC), 1))
        tlanes = jax.lax.broadcasted_iota(jnp.int32, (1, TOPK), 1)
        bvals = jnp.zeros((1, TOPK), jnp.float32)
        bidx = jnp.zeros((1, TOPK), jnp.int32)
        for t in range(TOPK):
            m1 = jnp.max(cur, axis=1, keepdims=True)
            am1 = jnp.min(jnp.where(cur == m1, gidx, NP),
                          axis=1, keepdims=True)
            mval = jnp.max(m1, axis=0, keepdims=True)
            midx = jnp.min(jnp.where(m1 == mval, am1, NP),
                           axis=0, keepdims=True)
            bvals = jnp.where(tlanes == t, mval, bvals)
            bidx = jnp.where(tlanes == t, midx, bidx)
            cur = jnp.where(gidx == midx, -jnp.inf, cur)
        bests_ref[...] = bvals
        idx_ref[...] = bidx


def kernel(input, Wq, bq, Wk, bk):
    patches = input.reshape(H // S, S, W // S, S * C)
    patches = patches.transpose(0, 2, 1, 3).reshape(NP, PDIM)
    colsum, bests, idx = pl.pallas_call(
        _fused_kernel,
        grid=(NBLK,),
        in_specs=[
            pl.BlockSpec((NP, PDIM), lambda i: (0, 0)),
            pl.BlockSpec((PDIM, QD), lambda i: (0, 0)),
            pl.BlockSpec((1, QD), lambda i: (0, 0)),
            pl.BlockSpec((PDIM, KD), lambda i: (0, 0)),
            pl.BlockSpec((1, KD), lambda i: (0, 0)),
        ],
        out_specs=[
            pl.BlockSpec((1, NP), lambda i: (0, 0)),
            pl.BlockSpec((1, TOPK), lambda i: (0, 0)),
            pl.BlockSpec((1, TOPK), lambda i: (0, 0)),
        ],
        out_shape=[
            jax.ShapeDtypeStruct((1, NP), jnp.float32),
            jax.ShapeDtypeStruct((1, TOPK), jnp.float32),
            jax.ShapeDtypeStruct((1, TOPK), jnp.int32),
        ],
        scratch_shapes=[pltpu.VMEM((NP, KD), jnp.float32)],
    )(patches, Wq, bq.reshape(1, QD), Wk, bk.reshape(1, KD))
    return bests[0], idx[0], colsum[0]


# speedup vs baseline: 1.0303x; 1.0023x over previous
import jax
import jax.numpy as jnp
from jax.experimental import pallas as pl
from jax.experimental.pallas import tpu as pltpu

H, W, C = 512, 512, 3
S = 8
QD, KD = 32, 32
TOPK = 64
NP = (H // S) * (W // S)
PDIM = S * S * C
RB = 1024
NBLK = NP // RB


def _fused_kernel(patches_ref, wq_ref, bq_ref, wk_ref, bk_ref,
                  colsum_ref, bests_ref, idx_ref, k_scratch):
    i = pl.program_id(0)

    @pl.when(i == 0)
    def _init():
        k_scratch[...] = (
            jnp.dot(patches_ref[...], wk_ref[...],
                    preferred_element_type=jnp.float32) + bk_ref[...])
        colsum_ref[...] = jnp.zeros_like(colsum_ref)

    p_blk = patches_ref[pl.ds(i * RB, RB), :]
    q = (jnp.dot(p_blk, wq_ref[...], preferred_element_type=jnp.float32)
         + bq_ref[...])
    s = jax.lax.dot_general(
        q, k_scratch[...], (((1,), (1,)), ((), ())),
        preferred_element_type=jnp.float32)
    kk = k_scratch[...]
    mk2 = jnp.max(jnp.sum(kk * kk, axis=1, keepdims=True))
    b = jnp.sqrt(jnp.sum(q * q, axis=1, keepdims=True) * mk2)
    p = jnp.exp(s - b)
    z = jnp.sum(p, axis=1, keepdims=True)
    colsum_ref[...] += jnp.sum(p / z, axis=0, keepdims=True)

    @pl.when(i == NBLK - 1)
    def _topk():
        FR = 8
        FC = NP // FR
        cur = colsum_ref[...].reshape(FR, FC)
        gidx = (jax.lax.broadcasted_iota(jnp.int32, (FR, FC), 0) * FC
                + jax.lax.broadcasted_iota(jnp.int32, (FR, FC), 1))
        tlanes = jax.lax.broadcasted_iota(jnp.int32, (1, TOPK), 1)
        bvals = jnp.zeros((1, TOPK), jnp.float32)
        bidx = jnp.zeros((1, TOPK), jnp.int32)
        for t in range(TOPK):
            m1 = jnp.max(cur, axis=1, keepdims=True)
            am1 = jnp.min(jnp.where(cur == m1, gidx, NP),
                          axis=1, keepdims=True)
            mval = jnp.max(m1, axis=0, keepdims=True)
            midx = jnp.min(jnp.where(m1 == mval, am1, NP),
                           axis=0, keepdims=True)
            bvals = jnp.where(tlanes == t, mval, bvals)
            bidx = jnp.where(tlanes == t, midx, bidx)
            cur = jnp.where(gidx == midx, -jnp.inf, cur)
        bests_ref[...] = bvals
        idx_ref[...] = bidx


def kernel(input, Wq, bq, Wk, bk):
    patches = input.reshape(H // S, S, W // S, S * C)
    patches = patches.transpose(0, 2, 1, 3).reshape(NP, PDIM)
    colsum, bests, idx = pl.pallas_call(
        _fused_kernel,
        grid=(NBLK,),
        in_specs=[
            pl.BlockSpec((NP, PDIM), lambda i: (0, 0)),
            pl.BlockSpec((PDIM, QD), lambda i: (0, 0)),
            pl.BlockSpec((1, QD), lambda i: (0, 0)),
            pl.BlockSpec((PDIM, KD), lambda i: (0, 0)),
            pl.BlockSpec((1, KD), lambda i: (0, 0)),
        ],
        out_specs=[
            pl.BlockSpec((1, NP), lambda i: (0, 0)),
            pl.BlockSpec((1, TOPK), lambda i: (0, 0)),
            pl.BlockSpec((1, TOPK), lambda i: (0, 0)),
        ],
        out_shape=[
            jax.ShapeDtypeStruct((1, NP), jnp.float32),
            jax.ShapeDtypeStruct((1, TOPK), jnp.float32),
            jax.ShapeDtypeStruct((1, TOPK), jnp.int32),
        ],
        scratch_shapes=[pltpu.VMEM((NP, KD), jnp.float32)],
    )(patches, Wq, bq.reshape(1, QD), Wk, bk.reshape(1, KD))
    return bests[0], idx[0], colsum[0]
